# baseline (speedup 1.0000x reference)
import jax
import jax.numpy as jnp
from jax import lax
from jax.experimental import pallas as pl
from jax.experimental.pallas import tpu as pltpu
from jax.experimental.pallas import tpu_sc as plsc

_MAX_SEQ_LEN = 8192
_D_MODEL = 1024
_NUM_WORKERS = 32
_ROWS_PER_WORKER = _MAX_SEQ_LEN // _NUM_WORKERS


def _copy_body(table_hbm, out_hbm):
    wid = lax.axis_index("s") * 2 + lax.axis_index("c")
    base = wid * _ROWS_PER_WORKER
    pltpu.sync_copy(
        table_hbm.at[pl.ds(base, _ROWS_PER_WORKER)],
        out_hbm.at[pl.ds(base, _ROWS_PER_WORKER)],
    )


def kernel(x, table):
    mesh = plsc.VectorSubcoreMesh(core_axis_name="c", subcore_axis_name="s")
    out = pl.kernel(
        _copy_body,
        out_type=jax.ShapeDtypeStruct((_MAX_SEQ_LEN, _D_MODEL), jnp.float32),
        mesh=mesh,
    )(table)
    return out[None]


# speedup vs baseline: 24.2688x; 24.2688x over previous
import jax
import jax.numpy as jnp
from jax import lax
from jax.experimental import pallas as pl
from jax.experimental.pallas import tpu as pltpu
from jax.experimental.pallas import tpu_sc as plsc

_MAX_SEQ_LEN = 8192
_D_MODEL = 1024
_NUM_WORKERS = 32
_ROWS_PER_WORKER = _MAX_SEQ_LEN // _NUM_WORKERS
_CHUNK = 16
_NCH = _ROWS_PER_WORKER // _CHUNK
_NBUF = 4


def _copy_body(table_hbm, out_hbm, *scr):
    bufs = scr[:_NBUF]
    in_sems = scr[_NBUF:2 * _NBUF]
    out_sems = scr[2 * _NBUF:3 * _NBUF]
    wid = lax.axis_index("s") * 2 + lax.axis_index("c")
    base = wid * _ROWS_PER_WORKER

    def in_copy(i):
        b = i % _NBUF
        return pltpu.make_async_copy(
            table_hbm.at[pl.ds(base + i * _CHUNK, _CHUNK)], bufs[b], in_sems[b])

    def out_copy(i):
        b = i % _NBUF
        return pltpu.make_async_copy(
            bufs[b], out_hbm.at[pl.ds(base + i * _CHUNK, _CHUNK)], out_sems[b])

    for i in range(_NBUF - 1):
        in_copy(i).start()
    for i in range(_NCH):
        nxt = i + _NBUF - 1
        if nxt < _NCH:
            if nxt >= _NBUF:
                out_copy(nxt - _NBUF).wait()
            in_copy(nxt).start()
        in_copy(i).wait()
        out_copy(i).start()
    for i in range(_NCH - _NBUF, _NCH):
        out_copy(i).wait()


def kernel(x, table):
    mesh = plsc.VectorSubcoreMesh(core_axis_name="c", subcore_axis_name="s")
    out = pl.kernel(
        _copy_body,
        out_type=jax.ShapeDtypeStruct((_MAX_SEQ_LEN, _D_MODEL), jnp.float32),
        scratch_types=(
            [pltpu.VMEM((_CHUNK, _D_MODEL), jnp.float32) for _ in range(_NBUF)]
            + [pltpu.SemaphoreType.DMA for _ in range(2 * _NBUF)]
        ),
        mesh=mesh,
    )(table)
    return out[None]


# speedup vs baseline: 24.8672x; 1.0247x over previous
import jax
import jax.numpy as jnp
from jax import lax
from jax.experimental import pallas as pl
from jax.experimental.pallas import tpu as pltpu
from jax.experimental.pallas import tpu_sc as plsc

_MAX_SEQ_LEN = 8192
_D_MODEL = 1024
_NUM_WORKERS = 32
_ROWS_PER_WORKER = _MAX_SEQ_LEN // _NUM_WORKERS
_CHUNK = 32
_NCH = _ROWS_PER_WORKER // _CHUNK
_NBUF = 3


def _copy_body(table_hbm, out_hbm, *scr):
    bufs = scr[:_NBUF]
    in_sems = scr[_NBUF:2 * _NBUF]
    out_sems = scr[2 * _NBUF:3 * _NBUF]
    wid = lax.axis_index("s") * 2 + lax.axis_index("c")
    base = wid * _ROWS_PER_WORKER

    def in_copy(i):
        b = i % _NBUF
        return pltpu.make_async_copy(
            table_hbm.at[pl.ds(base + i * _CHUNK, _CHUNK)], bufs[b], in_sems[b])

    def out_copy(i):
        b = i % _NBUF
        return pltpu.make_async_copy(
            bufs[b], out_hbm.at[pl.ds(base + i * _CHUNK, _CHUNK)], out_sems[b])

    for i in range(_NBUF - 1):
        in_copy(i).start()
    for i in range(_NCH):
        nxt = i + _NBUF - 1
        if nxt < _NCH:
            if nxt >= _NBUF:
                out_copy(nxt - _NBUF).wait()
            in_copy(nxt).start()
        in_copy(i).wait()
        out_copy(i).start()
    for i in range(_NCH - _NBUF, _NCH):
        out_copy(i).wait()


def kernel(x, table):
    mesh = plsc.VectorSubcoreMesh(core_axis_name="c", subcore_axis_name="s")
    out = pl.kernel(
        _copy_body,
        out_type=jax.ShapeDtypeStruct((_MAX_SEQ_LEN, _D_MODEL), jnp.float32),
        scratch_types=(
            [pltpu.VMEM((_CHUNK, _D_MODEL), jnp.float32) for _ in range(_NBUF)]
            + [pltpu.SemaphoreType.DMA for _ in range(2 * _NBUF)]
        ),
        mesh=mesh,
    )(table)
    return out[None]


# speedup vs baseline: 41.9810x; 1.6882x over previous
import jax
import jax.numpy as jnp
from jax.experimental import pallas as pl
from jax.experimental.pallas import tpu as pltpu

_MAX_SEQ_LEN = 8192
_D_MODEL = 1024
_BLK = 512


def _tc_body(in_ref, out_ref):
    out_ref[...] = in_ref[...]


def kernel(x, table):
    out = pl.pallas_call(
        _tc_body,
        grid=(_MAX_SEQ_LEN // _BLK,),
        in_specs=[pl.BlockSpec((_BLK, _D_MODEL), lambda i: (i, 0))],
        out_specs=pl.BlockSpec((_BLK, _D_MODEL), lambda i: (i, 0)),
        out_shape=jax.ShapeDtypeStruct((_MAX_SEQ_LEN, _D_MODEL), jnp.float32),
    )(table)
    return out[None]
